# baseline (speedup 1.0000x reference)
import functools

import jax
import jax.numpy as jnp
from jax import lax
from jax.experimental import pallas as pl
from jax.experimental.pallas import tpu as pltpu
from jax.experimental.pallas import tpu_sc as plsc

NC = 2
NS = 16
NW = NC * NS
L = 16

HIDDEN = 768
HS = HIDDEN // L
EPS = 1e-12
CHUNK = 64
ROWU = 8
NACC = 4


def _rsqrt(v):
    i = plsc.bitcast(v, jnp.int32)
    i = jnp.int32(0x5F3759DF) - lax.shift_right_logical(i, 1)
    y = plsc.bitcast(i, jnp.float32)
    for _ in range(2):
        y = y * (1.5 - 0.5 * v * y * y)
    return y


def _allsum(v, perms):
    dnums = lax.GatherDimensionNumbers(
        offset_dims=(), collapsed_slice_dims=(0,), start_index_map=(0,))
    for p in perms:
        v = v + lax.gather(
            v, p[:, None], dnums, slice_sizes=(1,),
            mode=lax.GatherScatterMode.PROMISE_IN_BOUNDS)
    return v


def _layernorm_chunk(buf, n_rows, gam_v, bet_v, plain_affine):
    inv_h = jnp.float32(1.0 / HIDDEN)

    lane = jnp.arange(L, dtype=jnp.int32)
    perms = [lane ^ k for k in (8, 4, 2, 1)]

    @plsc.parallel_loop(0, n_rows // ROWU)
    def group_body(g):
        r0 = g * ROWU
        mean_vs, inv_vs = [], []
        for j in range(ROWU):
            r = r0 + j
            acc = [None] * NACC
            acc2 = [None] * NACC
            for h in range(HS):
                x = buf[r, pl.ds(h * L, L)]
                if h < NACC:
                    acc[h] = x
                    acc2[h] = x * x
                else:
                    acc[h % NACC] = acc[h % NACC] + x
                    acc2[h % NACC] = acc2[h % NACC] + x * x
            s = (acc[0] + acc[1]) + (acc[2] + acc[3])
            s2 = (acc2[0] + acc2[1]) + (acc2[2] + acc2[3])
            mean = _allsum(s, perms) * inv_h
            var = jnp.maximum(
                _allsum(s2, perms) * inv_h - mean * mean, 0.0)
            mean_vs.append(mean)
            inv_vs.append(_rsqrt(var + EPS))

        @pl.when(plain_affine)
        def _():
            for j in range(ROWU):
                m, a = mean_vs[j], inv_vs[j]
                for h in range(HS):
                    sl = pl.ds(h * L, L)
                    x = buf[r0 + j, sl]
                    buf[r0 + j, sl] = (x - m) * a

        @pl.when(jnp.logical_not(plain_affine))
        def _():
            for j in range(ROWU):
                m, a = mean_vs[j], inv_vs[j]
                r = r0 + j

                def body(h, carry, m=m, a=a, r=r):
                    sl = pl.ds(h * L, L)
                    x = buf[r, sl]
                    buf[r, sl] = (x - m) * a * gam_v[sl] + bet_v[sl]
                    return carry

                lax.fori_loop(0, HS, body, 0)


def _chunk_sizes(tpw):
    if tpw % CHUNK == 0 and tpw >= 3 * CHUNK:
        mid = (tpw - 128) // CHUNK
        return [16, 48] + [CHUNK] * mid + [48, 16]
    return [CHUNK] * (tpw // CHUNK)


def _make_sc_call(batch, seq):
    n_tokens = batch * seq
    tpw = n_tokens // NW
    sizes = _chunk_sizes(tpw)
    offs = [sum(sizes[:i]) for i in range(len(sizes))]
    nchunk = len(sizes)
    mesh = plsc.VectorSubcoreMesh(
        core_axis_name="c", subcore_axis_name="s",
        num_cores=NC, num_subcores=NS)

    @functools.partial(
        pl.kernel,
        out_type=jax.ShapeDtypeStruct((batch, seq, HIDDEN), jnp.float32),
        mesh=mesh,
        compiler_params=pltpu.CompilerParams(
            needs_layout_passes=False, disable_bounds_checks=True),
        scratch_types=[
            pltpu.VMEM((tpw,), jnp.int32),
            pltpu.VMEM((CHUNK, HIDDEN), jnp.float32),
            pltpu.VMEM((CHUNK, HIDDEN), jnp.float32),
            pltpu.VMEM((HIDDEN,), jnp.float32),
            pltpu.VMEM((HIDDEN,), jnp.float32),
            pltpu.SemaphoreType.DMA,
            pltpu.SemaphoreType.DMA,
            pltpu.SemaphoreType.DMA,
            pltpu.SemaphoreType.DMA,
            pltpu.SemaphoreType.DMA,
        ],
    )
    def sc_call(ids_hbm, table_hbm, gam_hbm, bet_hbm, out_hbm,
                ids_v, rows_a, rows_b, gam_v, bet_v,
                gsem_a, gsem_b, osem_a, osem_b, gbsem):
        wid = lax.axis_index("s") * NC + lax.axis_index("c")
        base = wid * tpw
        b_idx = base // seq
        s0 = base % seq

        pltpu.sync_copy(ids_hbm.at[b_idx, pl.ds(s0, tpw)], ids_v)

        bufs = [rows_a, rows_b]
        gsems = [gsem_a, gsem_b]
        osems = [osem_a, osem_b]

        gathers = [
            pltpu.make_async_copy(
                table_hbm.at[ids_v.at[pl.ds(offs[c], sizes[c])]],
                bufs[c % 2].at[pl.ds(0, sizes[c])], gsems[c % 2])
            for c in range(nchunk)
        ]
        out_copies = []
        gathers[0].start()

        gb_g = pltpu.make_async_copy(gam_hbm, gam_v, gbsem)
        gb_b = pltpu.make_async_copy(bet_hbm, bet_v, gbsem)
        gb_g.start()
        gb_b.start()
        gb_g.wait()
        gb_b.wait()

        dev = jnp.zeros((L,), jnp.float32)
        for h in range(HS):
            sl = pl.ds(h * L, L)
            dev = dev + jnp.abs(gam_v[sl] - 1.0) + jnp.abs(bet_v[sl])
        plain_affine = jnp.sum(dev) == 0.0
        for c in range(nchunk):
            if c + 1 < nchunk:
                if c >= 1:
                    out_copies[c - 1].wait()
                gathers[c + 1].start()
            gathers[c].wait()
            _layernorm_chunk(
                bufs[c % 2], sizes[c], gam_v, bet_v, plain_affine)
            oc = pltpu.make_async_copy(
                bufs[c % 2].at[pl.ds(0, sizes[c])],
                out_hbm.at[b_idx, pl.ds(s0 + offs[c], sizes[c])],
                osems[c % 2])
            oc.start()
            out_copies.append(oc)
        for c in range(max(0, nchunk - 2), nchunk):
            out_copies[c].wait()

    return sc_call


def kernel(input_ids, token_type_ids, word_embeddings, ln_gamma, ln_beta):
    b, s = input_ids.shape
    return _make_sc_call(b, s)(
        input_ids, word_embeddings, ln_gamma, ln_beta)
